# speedup vs baseline: 3.9878x; 3.9878x over previous
import functools

import jax
import jax.numpy as jnp
from jax import lax
from jax.experimental import pallas as pl
from jax.experimental.pallas import tpu as pltpu
from jax.experimental.pallas import tpu_sc as plsc

D_HID = 128
ROWS_PER_GROUP = 128


@functools.partial(jax.jit, static_argnames=("batch", "seq"))
def _gather_rows(idx_flat, table, batch, seq):
    info = plsc.get_sparse_core_info()
    nc, ns = info.num_cores, info.num_subcores
    nw = nc * ns
    b_total = batch * seq
    b_per_w = b_total // nw
    n_groups = b_per_w // ROWS_PER_GROUP
    mesh = plsc.VectorSubcoreMesh(core_axis_name="c", subcore_axis_name="s")

    @functools.partial(
        pl.kernel,
        mesh=mesh,
        out_type=jax.ShapeDtypeStruct((b_total, D_HID), jnp.float32),
        scratch_types=[
            pltpu.VMEM((b_per_w,), jnp.int32),
            pltpu.VMEM((ROWS_PER_GROUP, D_HID), jnp.float32),
            pltpu.SemaphoreType.DMA,
        ],
    )
    def sc_kernel(idx_hbm, table_hbm, out_hbm, idx_v, rows_v, gsem):
        wid = lax.axis_index("s") * nc + lax.axis_index("c")
        base = wid * b_per_w
        pltpu.sync_copy(idx_hbm.at[pl.ds(base, b_per_w)], idx_v)

        def body(g, carry):
            off = pl.multiple_of(g * ROWS_PER_GROUP, 8)
            pltpu.async_copy(
                table_hbm.at[idx_v.at[pl.ds(off, ROWS_PER_GROUP)]],
                rows_v,
                gsem,
            ).wait()
            pltpu.sync_copy(
                rows_v,
                out_hbm.at[pl.ds(base + off, ROWS_PER_GROUP)],
            )
            return carry

        lax.fori_loop(0, n_groups, body, 0)

    return sc_kernel(idx_flat, table)


def kernel(doys, pos_table):
    batch, seq = doys.shape
    idx_flat = doys.astype(jnp.int32).reshape(batch * seq)
    out = _gather_rows(idx_flat, pos_table, batch, seq)
    return out.reshape(batch, seq, D_HID)


# speedup vs baseline: 4.0700x; 1.0206x over previous
import functools

import jax
import jax.numpy as jnp
from jax import lax
from jax.experimental import pallas as pl
from jax.experimental.pallas import tpu as pltpu
from jax.experimental.pallas import tpu_sc as plsc

D_HID = 128
G_ROWS = 128
CHUNK = 256
GP = CHUNK // G_ROWS


@functools.partial(jax.jit, static_argnames=("batch", "seq"))
def _gather_rows(idx_flat, table, batch, seq):
    info = plsc.get_sparse_core_info()
    nc, ns = info.num_cores, info.num_subcores
    nw = nc * ns
    b_total = batch * seq
    b_per_w = b_total // nw
    n_chunks = b_per_w // CHUNK
    mesh = plsc.VectorSubcoreMesh(core_axis_name="c", subcore_axis_name="s")

    @functools.partial(
        pl.kernel,
        mesh=mesh,
        out_type=jax.ShapeDtypeStruct((b_total, D_HID), jnp.float32),
        scratch_types=[
            pltpu.VMEM((b_per_w,), jnp.int32),
            pltpu.VMEM((CHUNK, D_HID), jnp.float32),
            pltpu.VMEM((CHUNK, D_HID), jnp.float32),
            pltpu.SemaphoreType.DMA,
            pltpu.SemaphoreType.DMA,
            pltpu.SemaphoreType.DMA,
            pltpu.SemaphoreType.DMA,
        ],
    )
    def sc_kernel(idx_hbm, table_hbm, out_hbm, idx_v,
                  rows0, rows1, gsem0, gsem1, ssem0, ssem1):
        wid = lax.axis_index("s") * nc + lax.axis_index("c")
        base = wid * b_per_w
        pltpu.sync_copy(idx_hbm.at[pl.ds(base, b_per_w)], idx_v)

        bufs = ((rows0, gsem0, ssem0), (rows1, gsem1, ssem1))

        def gather(c, rows, gsem):
            for j in range(GP):
                off = pl.multiple_of(c * CHUNK + j * G_ROWS, 8)
                pltpu.async_copy(
                    table_hbm.at[idx_v.at[pl.ds(off, G_ROWS)]],
                    rows.at[pl.ds(j * G_ROWS, G_ROWS)],
                    gsem,
                )

        def wait_gather(rows, gsem):
            pltpu.make_async_copy(out_hbm.at[pl.ds(0, CHUNK)], rows, gsem).wait()

        def scatter(c, rows, ssem):
            off = pl.multiple_of(base + c * CHUNK, 8)
            pltpu.async_copy(rows, out_hbm.at[pl.ds(off, CHUNK)], ssem)

        def wait_scatter(rows, ssem):
            pltpu.make_async_copy(rows, out_hbm.at[pl.ds(0, CHUNK)], ssem).wait()

        gather(0, rows0, gsem0)

        def body(i, carry):
            for b in range(2):
                c = 2 * i + b
                rows, gsem, ssem = bufs[b]
                orows, ogsem, ossem = bufs[1 - b]
                wait_gather(rows, gsem)

                @pl.when(c >= 1)
                def _():
                    wait_scatter(orows, ossem)

                @pl.when(c + 1 < n_chunks)
                def _():
                    gather(c + 1, orows, ogsem)

                scatter(c, rows, ssem)
            return carry

        lax.fori_loop(0, n_chunks // 2, body, 0)

        wait_scatter(rows1, ssem1)

    return sc_kernel(idx_flat, table)


def kernel(doys, pos_table):
    batch, seq = doys.shape
    idx_flat = doys.astype(jnp.int32).reshape(batch * seq)
    out = _gather_rows(idx_flat, pos_table, batch, seq)
    return out.reshape(batch, seq, D_HID)


# speedup vs baseline: 15.3934x; 3.7821x over previous
import functools

import jax
import jax.numpy as jnp
from jax import lax
from jax.experimental import pallas as pl
from jax.experimental.pallas import tpu as pltpu
from jax.experimental.pallas import tpu_sc as plsc

D_HID = 128
N_POS = 365
G_ROWS = 128
CHUNK = 256
GP = CHUNK // G_ROWS


@functools.partial(jax.jit, static_argnames=("batch", "seq"))
def _gather_rows(idx_flat, table, batch, seq):
    info = plsc.get_sparse_core_info()
    nc, ns = info.num_cores, info.num_subcores
    nw = nc * ns
    b_total = batch * seq
    b_per_w = b_total // nw
    n_chunks = b_per_w // CHUNK
    mesh = plsc.VectorSubcoreMesh(core_axis_name="c", subcore_axis_name="s")

    @functools.partial(
        pl.kernel,
        mesh=mesh,
        out_type=jax.ShapeDtypeStruct((b_total, D_HID), jnp.float32),
        scratch_types=[
            pltpu.VMEM((b_per_w,), jnp.int32),
            pltpu.VMEM((CHUNK, D_HID), jnp.float32),
            pltpu.VMEM((CHUNK, D_HID), jnp.float32),
            pltpu.VMEM_SHARED((N_POS, D_HID), jnp.float32),
            pltpu.SemaphoreType.DMA,
            pltpu.SemaphoreType.DMA,
            pltpu.SemaphoreType.DMA,
            pltpu.SemaphoreType.DMA,
        ],
    )
    def sc_kernel(idx_hbm, table_hbm, out_hbm, idx_v,
                  rows0, rows1, table_sp, gsem0, gsem1, ssem0, ssem1):
        wid = lax.axis_index("s") * nc + lax.axis_index("c")
        base = wid * b_per_w

        @pl.when(lax.axis_index("s") == 0)
        def _():
            pltpu.sync_copy(table_hbm, table_sp)

        plsc.subcore_barrier()
        pltpu.sync_copy(idx_hbm.at[pl.ds(base, b_per_w)], idx_v)

        bufs = ((rows0, gsem0, ssem0), (rows1, gsem1, ssem1))

        def gather(c, rows, gsem):
            for j in range(GP):
                off = pl.multiple_of(c * CHUNK + j * G_ROWS, 8)
                pltpu.async_copy(
                    table_sp.at[idx_v.at[pl.ds(off, G_ROWS)]],
                    rows.at[pl.ds(j * G_ROWS, G_ROWS)],
                    gsem,
                )

        def wait_gather(rows, gsem):
            pltpu.make_async_copy(out_hbm.at[pl.ds(0, CHUNK)], rows, gsem).wait()

        def scatter(c, rows, ssem):
            off = pl.multiple_of(base + c * CHUNK, 8)
            pltpu.async_copy(rows, out_hbm.at[pl.ds(off, CHUNK)], ssem)

        def wait_scatter(rows, ssem):
            pltpu.make_async_copy(rows, out_hbm.at[pl.ds(0, CHUNK)], ssem).wait()

        gather(0, rows0, gsem0)

        def body(i, carry):
            for b in range(2):
                c = 2 * i + b
                rows, gsem, ssem = bufs[b]
                orows, ogsem, ossem = bufs[1 - b]
                wait_gather(rows, gsem)

                @pl.when(c >= 1)
                def _():
                    wait_scatter(orows, ossem)

                @pl.when(c + 1 < n_chunks)
                def _():
                    gather(c + 1, orows, ogsem)

                scatter(c, rows, ssem)
            return carry

        lax.fori_loop(0, n_chunks // 2, body, 0)

        wait_scatter(rows1, ssem1)

    return sc_kernel(idx_flat, table)


def kernel(doys, pos_table):
    batch, seq = doys.shape
    idx_flat = doys.astype(jnp.int32).reshape(batch * seq)
    out = _gather_rows(idx_flat, pos_table, batch, seq)
    return out.reshape(batch, seq, D_HID)


# speedup vs baseline: 15.7832x; 1.0253x over previous
import functools

import jax
import jax.numpy as jnp
from jax import lax
from jax.experimental import pallas as pl
from jax.experimental.pallas import tpu as pltpu
from jax.experimental.pallas import tpu_sc as plsc

D_HID = 128
N_POS = 365
CHUNK = 128
NBUF = 4


@functools.partial(jax.jit, static_argnames=("batch", "seq"))
def _gather_rows(idx_flat, table, batch, seq):
    info = plsc.get_sparse_core_info()
    nc, ns = info.num_cores, info.num_subcores
    nw = nc * ns
    b_total = batch * seq
    b_per_w = b_total // nw
    n_chunks = b_per_w // CHUNK
    mesh = plsc.VectorSubcoreMesh(core_axis_name="c", subcore_axis_name="s")

    @functools.partial(
        pl.kernel,
        mesh=mesh,
        out_type=jax.ShapeDtypeStruct((b_total, D_HID), jnp.float32),
        scratch_types=(
            [pltpu.VMEM((b_per_w,), jnp.int32)]
            + [pltpu.VMEM((CHUNK, D_HID), jnp.float32) for _ in range(NBUF)]
            + [pltpu.VMEM_SHARED((N_POS, D_HID), jnp.float32)]
            + [pltpu.SemaphoreType.DMA for _ in range(2 * NBUF)]
        ),
    )
    def sc_kernel(idx_hbm, table_hbm, out_hbm, idx_v, *rest):
        rows = rest[:NBUF]
        table_sp = rest[NBUF]
        gsems = rest[NBUF + 1:2 * NBUF + 1]
        ssems = rest[2 * NBUF + 1:]

        wid = lax.axis_index("s") * nc + lax.axis_index("c")
        base = wid * b_per_w

        @pl.when(lax.axis_index("s") == 0)
        def _():
            pltpu.sync_copy(table_hbm, table_sp)

        plsc.subcore_barrier()
        pltpu.sync_copy(idx_hbm.at[pl.ds(base, b_per_w)], idx_v)

        def gather(c, b):
            off = pl.multiple_of(c * CHUNK, 8)
            pltpu.async_copy(
                table_sp.at[idx_v.at[pl.ds(off, CHUNK)]], rows[b], gsems[b]
            )

        def wait_gather(b):
            pltpu.make_async_copy(
                out_hbm.at[pl.ds(0, CHUNK)], rows[b], gsems[b]
            ).wait()

        def scatter(c, b):
            off = pl.multiple_of(base + c * CHUNK, 8)
            pltpu.async_copy(rows[b], out_hbm.at[pl.ds(off, CHUNK)], ssems[b])

        def wait_scatter(b):
            pltpu.make_async_copy(
                rows[b], out_hbm.at[pl.ds(0, CHUNK)], ssems[b]
            ).wait()

        AHEAD = 2
        for b in range(AHEAD):
            gather(b, b)

        def body(i, carry):
            for b in range(NBUF):
                c = NBUF * i + b
                wait_gather(b)

                nb = (b + AHEAD) % NBUF

                @pl.when(c + AHEAD < n_chunks)
                def _():
                    @pl.when(c >= NBUF - AHEAD)
                    def _():
                        wait_scatter(nb)

                    gather(c + AHEAD, nb)

                scatter(c, b)
            return carry

        lax.fori_loop(0, n_chunks // NBUF, body, 0)

        for b in range(NBUF):
            wait_scatter(b)

    return sc_kernel(idx_flat, table)


def kernel(doys, pos_table):
    batch, seq = doys.shape
    idx_flat = doys.astype(jnp.int32).reshape(batch * seq)
    out = _gather_rows(idx_flat, pos_table, batch, seq)
    return out.reshape(batch, seq, D_HID)
